# Initial kernel scaffold; baseline (speedup 1.0000x reference)
import jax
import jax.numpy as jnp
from jax.experimental import pallas as pl


def kernel(x, We0, be0, We1, be1, Wg, Wn, Wh0, bh0, Wh1, bh1, Wh2, bh2):
    raise NotImplementedError("write your pallas kernel here")


# speedup vs baseline: 1.0835x; 1.0835x over previous
import functools

import jax
import jax.numpy as jnp
from jax.experimental import pallas as pl
from jax.experimental.pallas import tpu as pltpu

B = 4096
D = 2048
E = 16
H = 512
T = 2
TOPK = 2

_GATE_BB = 1024
_EXP_BB = 1024


def _gate_body(x_ref, wcat_ref, eps_ref, g_ref):
    x = x_ref[...]
    proj = jnp.dot(x, wcat_ref[...], preferred_element_type=jnp.float32)
    ii = jax.lax.broadcasted_iota(jnp.int32, (x.shape[0], E), 1)
    for t in range(T):
        mean = proj[:, t * E:(t + 1) * E]
        npj = proj[:, (T + t) * E:(T + t + 1) * E]
        std = jnp.maximum(npj, 0.0) + jnp.log1p(jnp.exp(-jnp.abs(npj)))
        noisy = mean + eps_ref[t] * std
        v1 = jnp.max(noisy, axis=1, keepdims=True)
        first1 = jnp.min(jnp.where(noisy == v1, ii, E), axis=1, keepdims=True)
        n2 = jnp.where(ii == first1, -jnp.inf, noisy)
        v2 = jnp.max(n2, axis=1, keepdims=True)
        routing = jnp.where(noisy < v2, -jnp.float32(1e30), noisy)
        ex = jnp.exp(routing - v1)
        g_ref[t] = ex / jnp.sum(ex, axis=1, keepdims=True)


def _expert_body(x_ref, we0_ref, be0_ref, we1_ref, be1_ref, g_ref, go_ref):
    e = pl.program_id(1)

    @pl.when(e == 0)
    def _():
        go_ref[...] = jnp.zeros_like(go_ref)

    h = jnp.maximum(
        jnp.dot(x_ref[...], we0_ref[0], preferred_element_type=jnp.float32)
        + be0_ref[0], 0.0)
    o = jnp.dot(h, we1_ref[0], preferred_element_type=jnp.float32) \
        + be1_ref[0]
    lane = jax.lax.broadcasted_iota(jnp.int32, (x_ref.shape[0], E), 1)
    for t in range(T):
        gcol = jnp.sum(jnp.where(lane == e, g_ref[t], 0.0), axis=1,
                       keepdims=True)
        go_ref[t] += gcol * o


def _head_body(go_ref, wh0_ref, bh0_ref, wh1_ref, bh1_ref, wh2_ref, bh2_ref,
               out_ref):
    a = jnp.maximum(
        jnp.dot(go_ref[0], wh0_ref[0], preferred_element_type=jnp.float32)
        + bh0_ref[0], 0.0)
    b = jnp.maximum(
        jnp.dot(a, wh1_ref[0], preferred_element_type=jnp.float32)
        + bh1_ref[0], 0.0)
    out_ref[0] = jnp.dot(b, wh2_ref[0], preferred_element_type=jnp.float32) \
        + bh2_ref[0]


def kernel(x, We0, be0, We1, be1, Wg, Wn, Wh0, bh0, Wh1, bh1, Wh2, bh2):
    eps_key = jax.random.key(42)
    eps = jnp.stack([
        jax.random.normal(jax.random.fold_in(eps_key, i), (B, E), jnp.float32)
        for i in range(T)])

    wcat = jnp.concatenate(
        [Wg[0], Wg[1], Wn[0], Wn[1]], axis=1)

    g = pl.pallas_call(
        _gate_body,
        grid=(B // _GATE_BB,),
        in_specs=[
            pl.BlockSpec((_GATE_BB, D), lambda i: (i, 0)),
            pl.BlockSpec((D, 4 * E), lambda i: (0, 0)),
            pl.BlockSpec((T, _GATE_BB, E), lambda i: (0, i, 0)),
        ],
        out_specs=pl.BlockSpec((T, _GATE_BB, E), lambda i: (0, i, 0)),
        out_shape=jax.ShapeDtypeStruct((T, B, E), jnp.float32),
    )(x, wcat, eps)

    go = pl.pallas_call(
        _expert_body,
        grid=(B // _EXP_BB, E),
        in_specs=[
            pl.BlockSpec((_EXP_BB, D), lambda i, e: (i, 0)),
            pl.BlockSpec((1, D, H), lambda i, e: (e, 0, 0)),
            pl.BlockSpec((1, 1, H), lambda i, e: (e, 0, 0)),
            pl.BlockSpec((1, H, H), lambda i, e: (e, 0, 0)),
            pl.BlockSpec((1, 1, H), lambda i, e: (e, 0, 0)),
            pl.BlockSpec((T, _EXP_BB, E), lambda i, e: (0, i, 0)),
        ],
        out_specs=pl.BlockSpec((T, _EXP_BB, H), lambda i, e: (0, i, 0)),
        out_shape=jax.ShapeDtypeStruct((T, B, H), jnp.float32),
    )(x, We0, be0[:, None, :], We1, be1[:, None, :], g)

    out = pl.pallas_call(
        _head_body,
        grid=(T,),
        in_specs=[
            pl.BlockSpec((1, B, H), lambda t: (t, 0, 0)),
            pl.BlockSpec((1, H, 512), lambda t: (t, 0, 0)),
            pl.BlockSpec((1, 1, 512), lambda t: (t, 0, 0)),
            pl.BlockSpec((1, 512, 256), lambda t: (t, 0, 0)),
            pl.BlockSpec((1, 1, 256), lambda t: (t, 0, 0)),
            pl.BlockSpec((1, 256, 1), lambda t: (t, 0, 0)),
            pl.BlockSpec((1, 1, 1), lambda t: (t, 0, 0)),
        ],
        out_specs=pl.BlockSpec((1, B, 1), lambda t: (t, 0, 0)),
        out_shape=jax.ShapeDtypeStruct((T, B, 1), jnp.float32),
    )(go, Wh0, bh0[:, None, :], Wh1, bh1[:, None, :], Wh2, bh2[:, None, :])
    return out
